# baseline (speedup 1.0000x reference)
import functools

import jax
import jax.numpy as jnp
from jax import lax
from jax.experimental import pallas as pl
from jax.experimental.pallas import tpu as pltpu
from jax.experimental.pallas import tpu_sc as plsc

N_NODES = 10000
N_EDGES = 320000
D_IN = 128
D_HID = 128
D_OUT = 2

NUM_CORES = 2
NUM_SUBCORES = 16
NW = NUM_CORES * NUM_SUBCORES
EDGES_PER_TILE = N_EDGES // NW
CHUNK = 128
FULL_CHUNKS = EDGES_PER_TILE // CHUNK
TAIL = EDGES_PER_TILE - FULL_CHUNKS * CHUNK
T2_LEN = 2 * N_NODES
VECS_PER_TILE = EDGES_PER_TILE // 16

_MESH = plsc.VectorSubcoreMesh(
    core_axis_name="c", subcore_axis_name="s",
    num_cores=NUM_CORES, num_subcores=NUM_SUBCORES)


@functools.partial(
    pl.kernel, mesh=_MESH,
    out_type=[
        jax.ShapeDtypeStruct((NUM_CORES, N_NODES, D_IN), jnp.float32),
        jax.ShapeDtypeStruct((NW, N_NODES), jnp.float32),
    ],
    scratch_types=[
        pltpu.VMEM((CHUNK,), jnp.int32),
        pltpu.VMEM((CHUNK,), jnp.int32),
        pltpu.VMEM((CHUNK,), jnp.int32),
        pltpu.VMEM((CHUNK,), jnp.int32),
        pltpu.VMEM((TAIL,), jnp.int32),
        pltpu.VMEM((TAIL,), jnp.int32),
        pltpu.VMEM((CHUNK, D_IN), jnp.float32),
        pltpu.VMEM((CHUNK, D_IN), jnp.float32),
        pltpu.VMEM((TAIL, D_IN), jnp.float32),
        pltpu.VMEM((N_NODES,), jnp.float32),
        pltpu.VMEM_SHARED((N_NODES, D_IN), jnp.float32),
        pltpu.SemaphoreType.DMA,
        pltpu.SemaphoreType.DMA,
        pltpu.SemaphoreType.DMA,
        pltpu.SemaphoreType.DMA,
    ],
    compiler_params=pltpu.CompilerParams(needs_layout_passes=False),
    name="sage_agg1")
def _agg1(x_hbm, src2_hbm, dst2_hbm, zerosf_hbm, zerosd_hbm,
          feat_out, deg_out, src_a, src_b, dst_a, dst_b, src_t, dst_t,
          rows_a, rows_b, rows_t, deg_v, accf_s,
          sem_ga, sem_gb, sem_ia, sem_ib):
    cid = lax.axis_index("c")
    sid = lax.axis_index("s")
    wid = cid * NUM_SUBCORES + sid

    @pl.when(sid < 5)
    def _init():
        seg = pl.ds(sid * 2000, 2000)
        pltpu.sync_copy(zerosf_hbm.at[seg], accf_s.at[seg])
    pltpu.sync_copy(zerosd_hbm, deg_v)
    plsc.subcore_barrier()

    ones16 = jnp.full((16,), 1.0, jnp.float32)

    def deg_chunk(dref, n):
        for k in range(n // 16):
            plsc.addupdate_scatter(deg_v, [dref[pl.ds(k * 16, 16)]], ones16)

    def fire_idx(j, sref, dref, sem):
        pltpu.async_copy(src2_hbm.at[wid, pl.ds(j * CHUNK, CHUNK)], sref, sem)
        pltpu.async_copy(dst2_hbm.at[wid, pl.ds(j * CHUNK, CHUNK)], dref, sem)

    def wait_idx(j, sref, dref, sem):
        pltpu.make_async_copy(src2_hbm.at[wid, pl.ds(j * CHUNK, CHUNK)],
                              sref, sem).wait()
        pltpu.make_async_copy(dst2_hbm.at[wid, pl.ds(j * CHUNK, CHUNK)],
                              dref, sem).wait()

    pltpu.sync_copy(src2_hbm.at[wid, pl.ds(0, CHUNK)], src_a)
    pltpu.sync_copy(dst2_hbm.at[wid, pl.ds(0, CHUNK)], dst_a)
    pltpu.async_copy(x_hbm.at[src_a], rows_a, sem_ga)
    fire_idx(1, src_b, dst_b, sem_ib)

    def body(i, carry):
        j0 = 2 * i
        wait_idx(j0 + 1, src_b, dst_b, sem_ib)
        pltpu.async_copy(x_hbm.at[src_b], rows_b, sem_gb)
        pltpu.make_async_copy(x_hbm.at[src_a], rows_a, sem_ga).wait()
        deg_chunk(dst_a, CHUNK)
        pltpu.sync_copy(rows_a, accf_s.at[dst_a], add=True)

        @pl.when(j0 + 2 < FULL_CHUNKS)
        def _fire_a():
            fire_idx(j0 + 2, src_a, dst_a, sem_ia)
        pltpu.make_async_copy(x_hbm.at[src_b], rows_b, sem_gb).wait()
        deg_chunk(dst_b, CHUNK)

        @pl.when(j0 + 2 < FULL_CHUNKS)
        def _gather_a():
            wait_idx(j0 + 2, src_a, dst_a, sem_ia)
            pltpu.async_copy(x_hbm.at[src_a], rows_a, sem_ga)
        pltpu.sync_copy(rows_b, accf_s.at[dst_b], add=True)

        @pl.when(j0 + 3 < FULL_CHUNKS)
        def _fire_b():
            fire_idx(j0 + 3, src_b, dst_b, sem_ib)
        return carry

    lax.fori_loop(0, FULL_CHUNKS // 2, body, 0)

    toff = FULL_CHUNKS * CHUNK
    pltpu.sync_copy(src2_hbm.at[wid, pl.ds(toff, TAIL)], src_t)
    pltpu.sync_copy(dst2_hbm.at[wid, pl.ds(toff, TAIL)], dst_t)
    pltpu.async_copy(x_hbm.at[src_t], rows_t, sem_ga).wait()
    deg_chunk(dst_t, TAIL)
    pltpu.sync_copy(rows_t, accf_s.at[dst_t], add=True)

    pltpu.sync_copy(deg_v, deg_out.at[wid])
    plsc.subcore_barrier()

    @pl.when(sid < 5)
    def _flush():
        seg = pl.ds(sid * 2000, 2000)
        pltpu.sync_copy(accf_s.at[seg], feat_out.at[cid, seg])


@functools.partial(
    pl.kernel, mesh=_MESH,
    out_type=jax.ShapeDtypeStruct((NW, T2_LEN), jnp.float32),
    scratch_types=[
        pltpu.VMEM((EDGES_PER_TILE,), jnp.int32),
        pltpu.VMEM((EDGES_PER_TILE,), jnp.int32),
        pltpu.VMEM((T2_LEN,), jnp.float32),
        pltpu.VMEM((T2_LEN,), jnp.float32),
    ],
    compiler_params=pltpu.CompilerParams(needs_layout_passes=False),
    name="sage_agg2")
def _agg2(t2a_hbm, t2b_hbm, srcf_hbm, dstf_hbm, zeros2_hbm,
          q_out, src_v, dst_v, t2_v, acc_v):
    cid = lax.axis_index("c")
    sid = lax.axis_index("s")
    wid = cid * NUM_SUBCORES + sid

    pltpu.sync_copy(t2a_hbm, t2_v.at[pl.ds(0, N_NODES)])
    pltpu.sync_copy(t2b_hbm, t2_v.at[pl.ds(N_NODES, N_NODES)])
    pltpu.sync_copy(zeros2_hbm, acc_v)
    pltpu.sync_copy(srcf_hbm.at[wid], src_v)
    pltpu.sync_copy(dstf_hbm.at[wid], dst_v)

    plane = jnp.full((16,), N_NODES, jnp.int32)

    def body(i, carry):
        s16 = src_v[pl.ds(i * 16, 16)]
        d16 = dst_v[pl.ds(i * 16, 16)]
        v0 = plsc.load_gather(t2_v, [s16])
        plsc.addupdate_scatter(acc_v, [d16], v0)
        v1 = plsc.load_gather(t2_v, [s16 + plane])
        plsc.addupdate_scatter(acc_v, [d16 + plane], v1)
        return carry

    lax.fori_loop(0, VECS_PER_TILE, body, 0)
    pltpu.sync_copy(acc_v, q_out.at[wid])


ROWS_BLK = 1000
N_BLKS = N_NODES // ROWS_BLK
PLANE_BLKS = N_NODES // ROWS_BLK


def _layer1_body(p0, p1, dpt, x, w1l, w1r, b1, w2cat, b2, t2a, t2b, t2r):
    deg = jnp.maximum(jnp.sum(dpt[...], axis=1, keepdims=True), 1.0)
    mean = (p0[...] + p1[...]) / deg
    hl = lax.dot_general(mean, w1l[...], (((1,), (1,)), ((), ())),
                         preferred_element_type=jnp.float32)
    hr = lax.dot_general(x[...], w1r[...], (((1,), (1,)), ((), ())),
                         preferred_element_type=jnp.float32)
    h = jnp.maximum(hl + hr + b1[...], 0.0)
    o = lax.dot_general(h, w2cat[...], (((1,), (1,)), ((), ())),
                        preferred_element_type=jnp.float32)
    t2a[...] = o[:, 0:1]
    t2b[...] = o[:, 1:2]
    t2r[...] = o[:, 2:4] + b2[...]


def _layer1_tc(p0, p1, dpt, x, w1l, w1r, b1, w2cat, b2):
    blk = lambda w: pl.BlockSpec((ROWS_BLK, w), lambda i: (i, 0))
    full = lambda a: pl.BlockSpec(a.shape, lambda i: (0,) * a.ndim)
    return pl.pallas_call(
        _layer1_body,
        grid=(N_BLKS,),
        in_specs=[blk(D_IN), blk(D_IN), blk(NW), blk(D_IN),
                  full(w1l), full(w1r), full(b1), full(w2cat), full(b2)],
        out_specs=[blk(1), blk(1), blk(D_OUT)],
        out_shape=[jax.ShapeDtypeStruct((N_NODES, 1), jnp.float32),
                   jax.ShapeDtypeStruct((N_NODES, 1), jnp.float32),
                   jax.ShapeDtypeStruct((N_NODES, D_OUT), jnp.float32)],
    )(p0, p1, dpt, x, w1l, w1r, b1, w2cat, b2)


def _final_body(qbt, dpt, t2r, out):
    deg = jnp.maximum(jnp.sum(dpt[...], axis=1, keepdims=True), 1.0)
    q0 = jnp.sum(qbt[:, :NW], axis=1, keepdims=True)
    q1 = jnp.sum(qbt[:, NW:], axis=1, keepdims=True)
    out[...] = jnp.concatenate([q0, q1], axis=1) / deg + t2r[...]


def _final_tc(qbt, dpt, t2r):
    blk = lambda w: pl.BlockSpec((ROWS_BLK, w), lambda i: (i, 0))
    return pl.pallas_call(
        _final_body,
        grid=(N_BLKS,),
        in_specs=[blk(2 * NW), blk(NW), blk(D_OUT)],
        out_specs=blk(D_OUT),
        out_shape=jax.ShapeDtypeStruct((N_NODES, D_OUT), jnp.float32),
    )(qbt, dpt, t2r)


def kernel(x, edge_index, W1_l, W1_r, b1, W2_l, W2_r, b2):
    src = edge_index[0].astype(jnp.int32)
    dst = edge_index[1].astype(jnp.int32)

    zeros_feat = jnp.zeros((N_NODES, D_IN), jnp.float32)
    zeros_deg = jnp.zeros((N_NODES,), jnp.float32)
    zeros_t2 = jnp.zeros((T2_LEN,), jnp.float32)

    w2cat = jnp.concatenate([W2_l, W2_r], axis=0)
    b2r = b2.reshape(1, D_OUT)

    src2 = src.reshape(NW, EDGES_PER_TILE)
    dst2 = dst.reshape(NW, EDGES_PER_TILE)

    feat_p, deg_p = _agg1(x, src2, dst2, zeros_feat, zeros_deg)
    dpt = deg_p.T

    t2a, t2b, t2r = _layer1_tc(feat_p[0], feat_p[1], dpt, x,
                               W1_l, W1_r, b1.reshape(1, D_HID), w2cat, b2r)

    q_p = _agg2(t2a.reshape(N_NODES), t2b.reshape(N_NODES),
                src2, dst2, zeros_t2)
    qbt = jnp.concatenate(
        [q_p[:, :N_NODES], q_p[:, N_NODES:]], axis=0).T

    return _final_tc(qbt, dpt, t2r)


# speedup vs baseline: 1.0354x; 1.0354x over previous
import functools

import jax
import jax.numpy as jnp
from jax import lax
from jax.experimental import pallas as pl
from jax.experimental.pallas import tpu as pltpu
from jax.experimental.pallas import tpu_sc as plsc

N_NODES = 10000
N_EDGES = 320000
D_IN = 128
D_HID = 128
D_OUT = 2

NUM_CORES = 2
NUM_SUBCORES = 16
NW = NUM_CORES * NUM_SUBCORES
EDGES_PER_TILE = N_EDGES // NW
CHUNK = 128
FULL_CHUNKS = EDGES_PER_TILE // CHUNK
TAIL = EDGES_PER_TILE - FULL_CHUNKS * CHUNK
T2_LEN = 2 * N_NODES
VECS_PER_TILE = EDGES_PER_TILE // 16

_MESH = plsc.VectorSubcoreMesh(
    core_axis_name="c", subcore_axis_name="s",
    num_cores=NUM_CORES, num_subcores=NUM_SUBCORES)


@functools.partial(
    pl.kernel, mesh=_MESH,
    out_type=[
        jax.ShapeDtypeStruct((NUM_CORES, N_NODES, D_IN), jnp.float32),
        jax.ShapeDtypeStruct((NW, N_NODES), jnp.float32),
    ],
    scratch_types=[
        pltpu.VMEM((CHUNK,), jnp.int32),
        pltpu.VMEM((CHUNK,), jnp.int32),
        pltpu.VMEM((CHUNK,), jnp.int32),
        pltpu.VMEM((CHUNK,), jnp.int32),
        pltpu.VMEM((TAIL,), jnp.int32),
        pltpu.VMEM((TAIL,), jnp.int32),
        pltpu.VMEM((CHUNK, D_IN), jnp.float32),
        pltpu.VMEM((CHUNK, D_IN), jnp.float32),
        pltpu.VMEM((TAIL, D_IN), jnp.float32),
        pltpu.VMEM((N_NODES,), jnp.float32),
        pltpu.VMEM_SHARED((N_NODES, D_IN), jnp.float32),
        pltpu.SemaphoreType.DMA,
        pltpu.SemaphoreType.DMA,
        pltpu.SemaphoreType.DMA,
        pltpu.SemaphoreType.DMA,
    ],
    compiler_params=pltpu.CompilerParams(needs_layout_passes=False),
    name="sage_agg1")
def _agg1(x_hbm, src2_hbm, dst2_hbm, zerosf_hbm, zerosd_hbm,
          feat_out, deg_out, src_a, src_b, dst_a, dst_b, src_t, dst_t,
          rows_a, rows_b, rows_t, deg_v, accf_s,
          sem_ga, sem_gb, sem_ia, sem_ib):
    cid = lax.axis_index("c")
    sid = lax.axis_index("s")
    wid = cid * NUM_SUBCORES + sid

    @pl.when(sid < 5)
    def _init():
        seg = pl.ds(sid * 2000, 2000)
        pltpu.sync_copy(zerosf_hbm.at[seg], accf_s.at[seg])
    pltpu.sync_copy(zerosd_hbm, deg_v)
    plsc.subcore_barrier()

    ones16 = jnp.full((16,), 1.0, jnp.float32)

    def deg_chunk(dref, n):
        for k in range(n // 16):
            plsc.addupdate_scatter(deg_v, [dref[pl.ds(k * 16, 16)]], ones16)

    def fire_idx(j, sref, dref, sem):
        pltpu.async_copy(src2_hbm.at[wid, pl.ds(j * CHUNK, CHUNK)], sref, sem)
        pltpu.async_copy(dst2_hbm.at[wid, pl.ds(j * CHUNK, CHUNK)], dref, sem)

    def wait_idx(j, sref, dref, sem):
        pltpu.make_async_copy(src2_hbm.at[wid, pl.ds(j * CHUNK, CHUNK)],
                              sref, sem).wait()
        pltpu.make_async_copy(dst2_hbm.at[wid, pl.ds(j * CHUNK, CHUNK)],
                              dref, sem).wait()

    pltpu.sync_copy(src2_hbm.at[wid, pl.ds(0, CHUNK)], src_a)
    pltpu.sync_copy(dst2_hbm.at[wid, pl.ds(0, CHUNK)], dst_a)
    pltpu.async_copy(x_hbm.at[src_a], rows_a, sem_ga)
    fire_idx(1, src_b, dst_b, sem_ib)

    def body(i, carry):
        j0 = 2 * i
        wait_idx(j0 + 1, src_b, dst_b, sem_ib)
        pltpu.async_copy(x_hbm.at[src_b], rows_b, sem_gb)
        pltpu.make_async_copy(x_hbm.at[src_a], rows_a, sem_ga).wait()
        deg_chunk(dst_a, CHUNK)
        pltpu.sync_copy(rows_a, accf_s.at[dst_a], add=True)

        @pl.when(j0 + 2 < FULL_CHUNKS)
        def _fire_a():
            fire_idx(j0 + 2, src_a, dst_a, sem_ia)
        pltpu.make_async_copy(x_hbm.at[src_b], rows_b, sem_gb).wait()
        deg_chunk(dst_b, CHUNK)

        @pl.when(j0 + 2 < FULL_CHUNKS)
        def _gather_a():
            wait_idx(j0 + 2, src_a, dst_a, sem_ia)
            pltpu.async_copy(x_hbm.at[src_a], rows_a, sem_ga)
        pltpu.sync_copy(rows_b, accf_s.at[dst_b], add=True)

        @pl.when(j0 + 3 < FULL_CHUNKS)
        def _fire_b():
            fire_idx(j0 + 3, src_b, dst_b, sem_ib)
        return carry

    lax.fori_loop(0, FULL_CHUNKS // 2, body, 0)

    toff = FULL_CHUNKS * CHUNK
    pltpu.sync_copy(src2_hbm.at[wid, pl.ds(toff, TAIL)], src_t)
    pltpu.sync_copy(dst2_hbm.at[wid, pl.ds(toff, TAIL)], dst_t)
    pltpu.async_copy(x_hbm.at[src_t], rows_t, sem_ga).wait()
    deg_chunk(dst_t, TAIL)
    pltpu.sync_copy(rows_t, accf_s.at[dst_t], add=True)

    pltpu.sync_copy(deg_v, deg_out.at[wid])
    plsc.subcore_barrier()

    @pl.when(sid < 5)
    def _flush():
        seg = pl.ds(sid * 2000, 2000)
        pltpu.sync_copy(accf_s.at[seg], feat_out.at[cid, seg])


@functools.partial(
    pl.kernel, mesh=_MESH,
    out_type=jax.ShapeDtypeStruct((NW, T2_LEN), jnp.float32),
    scratch_types=[
        pltpu.VMEM((EDGES_PER_TILE,), jnp.int32),
        pltpu.VMEM((EDGES_PER_TILE,), jnp.int32),
        pltpu.VMEM((T2_LEN,), jnp.float32),
        pltpu.VMEM((T2_LEN,), jnp.float32),
        pltpu.SemaphoreType.DMA,
    ],
    compiler_params=pltpu.CompilerParams(needs_layout_passes=False),
    name="sage_agg2")
def _agg2(t2a_hbm, t2b_hbm, srcf_hbm, dstf_hbm, zeros2_hbm,
          q_out, src_v, dst_v, t2_v, acc_v, sem):
    cid = lax.axis_index("c")
    sid = lax.axis_index("s")
    wid = cid * NUM_SUBCORES + sid

    pltpu.async_copy(t2a_hbm, t2_v.at[pl.ds(0, N_NODES)], sem)
    pltpu.async_copy(t2b_hbm, t2_v.at[pl.ds(N_NODES, N_NODES)], sem)
    pltpu.async_copy(zeros2_hbm, acc_v, sem)
    pltpu.async_copy(srcf_hbm.at[wid], src_v, sem)
    pltpu.async_copy(dstf_hbm.at[wid], dst_v, sem)
    pltpu.make_async_copy(t2a_hbm, t2_v.at[pl.ds(0, N_NODES)], sem).wait()
    pltpu.make_async_copy(t2b_hbm, t2_v.at[pl.ds(N_NODES, N_NODES)],
                          sem).wait()
    pltpu.make_async_copy(zeros2_hbm, acc_v, sem).wait()
    pltpu.make_async_copy(srcf_hbm.at[wid], src_v, sem).wait()
    pltpu.make_async_copy(dstf_hbm.at[wid], dst_v, sem).wait()

    plane = jnp.full((16,), N_NODES, jnp.int32)

    def body(i, carry):
        s16 = src_v[pl.ds(i * 16, 16)]
        d16 = dst_v[pl.ds(i * 16, 16)]
        v0 = plsc.load_gather(t2_v, [s16])
        plsc.addupdate_scatter(acc_v, [d16], v0)
        v1 = plsc.load_gather(t2_v, [s16 + plane])
        plsc.addupdate_scatter(acc_v, [d16 + plane], v1)
        return carry

    lax.fori_loop(0, VECS_PER_TILE, body, 0)
    pltpu.sync_copy(acc_v, q_out.at[wid])


ROWS_BLK = 1000
N_BLKS = N_NODES // ROWS_BLK
PLANE_BLKS = N_NODES // ROWS_BLK


def _layer1_body(p0, p1, dpt, x, w1l, w1r, b1, w2cat, b2, t2a, t2b, t2r):
    deg = jnp.maximum(jnp.sum(dpt[...], axis=1, keepdims=True), 1.0)
    mean = (p0[...] + p1[...]) / deg
    hl = lax.dot_general(mean, w1l[...], (((1,), (1,)), ((), ())),
                         preferred_element_type=jnp.float32)
    hr = lax.dot_general(x[...], w1r[...], (((1,), (1,)), ((), ())),
                         preferred_element_type=jnp.float32)
    h = jnp.maximum(hl + hr + b1[...], 0.0)
    o = lax.dot_general(h, w2cat[...], (((1,), (1,)), ((), ())),
                        preferred_element_type=jnp.float32)
    t2a[...] = o[:, 0:1]
    t2b[...] = o[:, 1:2]
    t2r[...] = o[:, 2:4] + b2[...]


def _layer1_tc(p0, p1, dpt, x, w1l, w1r, b1, w2cat, b2):
    blk = lambda w: pl.BlockSpec((ROWS_BLK, w), lambda i: (i, 0))
    full = lambda a: pl.BlockSpec(a.shape, lambda i: (0,) * a.ndim)
    return pl.pallas_call(
        _layer1_body,
        grid=(N_BLKS,),
        in_specs=[blk(D_IN), blk(D_IN), blk(NW), blk(D_IN),
                  full(w1l), full(w1r), full(b1), full(w2cat), full(b2)],
        out_specs=[blk(1), blk(1), blk(D_OUT)],
        out_shape=[jax.ShapeDtypeStruct((N_NODES, 1), jnp.float32),
                   jax.ShapeDtypeStruct((N_NODES, 1), jnp.float32),
                   jax.ShapeDtypeStruct((N_NODES, D_OUT), jnp.float32)],
    )(p0, p1, dpt, x, w1l, w1r, b1, w2cat, b2)


def _final_body(qbt, dpt, t2r, out):
    deg = jnp.maximum(jnp.sum(dpt[...], axis=1, keepdims=True), 1.0)
    q0 = jnp.sum(qbt[:, :NW], axis=1, keepdims=True)
    q1 = jnp.sum(qbt[:, NW:], axis=1, keepdims=True)
    out[...] = jnp.concatenate([q0, q1], axis=1) / deg + t2r[...]


def _final_tc(qbt, dpt, t2r):
    blk = lambda w: pl.BlockSpec((ROWS_BLK, w), lambda i: (i, 0))
    return pl.pallas_call(
        _final_body,
        grid=(N_BLKS,),
        in_specs=[blk(2 * NW), blk(NW), blk(D_OUT)],
        out_specs=blk(D_OUT),
        out_shape=jax.ShapeDtypeStruct((N_NODES, D_OUT), jnp.float32),
    )(qbt, dpt, t2r)


def kernel(x, edge_index, W1_l, W1_r, b1, W2_l, W2_r, b2):
    src = edge_index[0].astype(jnp.int32)
    dst = edge_index[1].astype(jnp.int32)

    zeros_feat = jnp.zeros((N_NODES, D_IN), jnp.float32)
    zeros_deg = jnp.zeros((N_NODES,), jnp.float32)
    zeros_t2 = jnp.zeros((T2_LEN,), jnp.float32)

    w2cat = jnp.concatenate([W2_l, W2_r], axis=0)
    b2r = b2.reshape(1, D_OUT)

    src2 = src.reshape(NW, EDGES_PER_TILE)
    dst2 = dst.reshape(NW, EDGES_PER_TILE)

    feat_p, deg_p = _agg1(x, src2, dst2, zeros_feat, zeros_deg)
    dpt = deg_p.T

    t2a, t2b, t2r = _layer1_tc(feat_p[0], feat_p[1], dpt, x,
                               W1_l, W1_r, b1.reshape(1, D_HID), w2cat, b2r)

    q_p = _agg2(t2a.reshape(N_NODES), t2b.reshape(N_NODES),
                src2, dst2, zeros_t2)
    qbt = jnp.concatenate(
        [q_p[:, :N_NODES], q_p[:, N_NODES:]], axis=0).T

    return _final_tc(qbt, dpt, t2r)
